# initial kernel scaffold (unmeasured)
import jax
import jax.numpy as jnp
from jax import lax
from jax.experimental import pallas as pl
from jax.experimental.pallas import tpu as pltpu


def kernel(
    x,
):
    def body(*refs):
        pass

    out_shape = jax.ShapeDtypeStruct(..., jnp.float32)
    return pl.pallas_call(body, out_shape=out_shape)(...)


# baseline (device time: 772217 ns/iter reference)
import functools

import jax
import jax.numpy as jnp
from jax import lax
from jax.experimental import pallas as pl
from jax.experimental.pallas import tpu as pltpu

Y = 4


def kernel(x):
    m, n = x.shape
    n_per = n // Y
    m_per = m

    def body(x_ref, out_ref, local_sem, send_sems, recv_sems):
        my_x = lax.axis_index("x")
        my_y = lax.axis_index("y")
        my_z = lax.axis_index("z")

        barrier = pltpu.get_barrier_semaphore()
        for d in range(1, Y):
            q = lax.rem(my_y + d, Y)
            pl.semaphore_signal(
                barrier, inc=1,
                device_id=(my_x, q, my_z),
                device_id_type=pl.DeviceIdType.MESH,
            )
        pl.semaphore_wait(barrier, Y - 1)

        local = pltpu.make_async_copy(
            x_ref.at[:, pl.ds(my_y * n_per, n_per)],
            out_ref.at[pl.ds(my_y * m_per, m_per), :],
            local_sem,
        )
        local.start()

        rdmas = []
        for d in range(1, Y):
            q = lax.rem(my_y + d, Y)
            rdma = pltpu.make_async_remote_copy(
                src_ref=x_ref.at[:, pl.ds(q * n_per, n_per)],
                dst_ref=out_ref.at[pl.ds(my_y * m_per, m_per), :],
                send_sem=send_sems.at[d - 1],
                recv_sem=recv_sems.at[d - 1],
                device_id=(my_x, q, my_z),
                device_id_type=pl.DeviceIdType.MESH,
            )
            rdma.start()
            rdmas.append(rdma)

        for rdma in rdmas:
            rdma.wait()
        local.wait()

        @functools.partial(
            pl.run_scoped, exit_sem=pltpu.SemaphoreType.REGULAR
        )
        def _(exit_sem):
            for d in range(1, Y):
                q = lax.rem(my_y + d, Y)
                pl.semaphore_signal(
                    exit_sem, inc=1,
                    device_id=(my_x, q, my_z),
                    device_id_type=pl.DeviceIdType.MESH,
                )
            pl.semaphore_wait(exit_sem, Y - 1)

    return pl.pallas_call(
        body,
        out_shape=jax.ShapeDtypeStruct((Y * m_per, n_per), x.dtype),
        in_specs=[pl.BlockSpec(memory_space=pltpu.MemorySpace.HBM)],
        out_specs=pl.BlockSpec(memory_space=pltpu.MemorySpace.HBM),
        scratch_shapes=[
            pltpu.SemaphoreType.DMA,
            pltpu.SemaphoreType.DMA((Y - 1,)),
            pltpu.SemaphoreType.DMA((Y - 1,)),
        ],
        compiler_params=pltpu.CompilerParams(collective_id=0),
    )(x)


# device time: 642783 ns/iter; 1.2014x vs baseline; 1.2014x over previous
import functools
import os

import jax
import jax.numpy as jnp
from jax import lax
from jax.experimental import pallas as pl
from jax.experimental.pallas import tpu as pltpu

Y = 4
QW = 256
M_PER = 4096
N_PER = 1024


def kernel(x):
    m, n = x.shape

    def body(x_ref, out_ref, local_sem,
             ysend, yrecv, xsend, xrecv, zsend, zrecv):
        my_x = lax.axis_index("x")
        my_y = lax.axis_index("y")
        my_z = lax.axis_index("z")
        a = lax.rem(my_z + 2 * my_x, 4)

        def sig(sem, dev):
            pl.semaphore_signal(
                sem, inc=1, device_id=dev,
                device_id_type=pl.DeviceIdType.MESH,
            )

        def barrier_with_partners(sem):
            for d in range(1, Y):
                q = lax.rem(my_y + d, Y)
                sig(sem, (my_x, q, my_z))
            sig(sem, (1 - my_x, my_y, my_z))

            @pl.when(my_z > 0)
            def _():
                sig(sem, (my_x, my_y, my_z - 1))

            @pl.when(my_z < 3)
            def _():
                sig(sem, (my_x, my_y, my_z + 1))

            pl.semaphore_wait(sem, 5)

            @pl.when((my_z > 0) & (my_z < 3))
            def _():
                pl.semaphore_wait(sem, 1)

        barrier_with_partners(pltpu.get_barrier_semaphore())

        local = pltpu.make_async_copy(
            x_ref.at[:, pl.ds(my_y * N_PER, N_PER)],
            out_ref.at[pl.ds(my_y * M_PER, M_PER), :],
            local_sem,
        )
        local.start()

        yrd = []
        for d in range(1, Y):
            qdev = lax.rem(my_y + d, Y)
            r = pltpu.make_async_remote_copy(
                src_ref=x_ref.at[:, pl.ds(qdev * N_PER + a * QW, QW)],
                dst_ref=out_ref.at[pl.ds(my_y * M_PER, M_PER),
                                   pl.ds(a * QW, QW)],
                send_sem=ysend.at[d - 1],
                recv_sem=yrecv.at[d - 1],
                device_id=(my_x, qdev, my_z),
                device_id_type=pl.DeviceIdType.MESH,
            )
            r.start()
            yrd.append(r)
        for r in yrd:
            r.wait()

        _PHASES = os.environ.get("A2A_PHASES", "yxz")
        if "x" in _PHASES:
            xrd = []
            for j in range(Y - 1):
                p = lax.rem(my_y + 1 + j, Y)
                r = pltpu.make_async_remote_copy(
                    src_ref=out_ref.at[pl.ds(p * M_PER, M_PER),
                                       pl.ds(a * QW, QW)],
                    dst_ref=out_ref.at[pl.ds(p * M_PER, M_PER),
                                       pl.ds(a * QW, QW)],
                    send_sem=xsend.at[j],
                    recv_sem=xrecv.at[j],
                    device_id=(1 - my_x, my_y, my_z),
                    device_id_type=pl.DeviceIdType.MESH,
                )
                r.start()
                xrd.append(r)
            for r in xrd:
                r.wait()

        def z_send(tz, quarter, qslot, sidx):
            rs = []
            for j in range(Y - 1):
                p = lax.rem(my_y + 1 + j, Y)
                r = pltpu.make_async_remote_copy(
                    src_ref=out_ref.at[pl.ds(p * M_PER, M_PER),
                                       pl.ds(quarter * QW, QW)],
                    dst_ref=out_ref.at[pl.ds(p * M_PER, M_PER),
                                       pl.ds(quarter * QW, QW)],
                    send_sem=zsend.at[sidx + j],
                    recv_sem=zrecv.at[qslot * 3 + j],
                    device_id=(my_x, my_y, tz),
                    device_id_type=pl.DeviceIdType.MESH,
                )
                r.start()
                rs.append(r)
            return rs

        if "z" in _PHASES:
            @pl.when(my_z == 0)
            def _():
                rs = z_send(1, 0, 0, 0)
                for r in rs:
                    r.wait_send()

            @pl.when(my_z == 1)
            def _():
                rs = z_send(0, 1, 0, 0)
                rs += z_send(0, 3, 1, 3)
                rs += z_send(2, 1, 0, 6)
                for r in rs:
                    r.wait_send()

            @pl.when(my_z == 2)
            def _():
                rs = z_send(1, 2, 1, 0)
                rs += z_send(3, 0, 0, 3)
                rs += z_send(3, 2, 1, 6)
                for r in rs:
                    r.wait_send()

            @pl.when(my_z == 3)
            def _():
                rs = z_send(2, 3, 1, 0)
                for r in rs:
                    r.wait_send()

            for qslot in range(2):
                for j in range(Y - 1):
                    p = lax.rem(my_y + 1 + j, Y)
                    r = pltpu.make_async_remote_copy(
                        src_ref=out_ref.at[pl.ds(0, M_PER), pl.ds(0, QW)],
                        dst_ref=out_ref.at[pl.ds(p * M_PER, M_PER),
                                           pl.ds(0, QW)],
                        send_sem=zsend.at[0],
                        recv_sem=zrecv.at[qslot * 3 + j],
                        device_id=(my_x, my_y, my_z),
                        device_id_type=pl.DeviceIdType.MESH,
                    )
                    r.wait_recv()

        local.wait()

        @functools.partial(
            pl.run_scoped, exit_sem=pltpu.SemaphoreType.REGULAR
        )
        def _(exit_sem):
            barrier_with_partners(exit_sem)

    return pl.pallas_call(
        body,
        out_shape=jax.ShapeDtypeStruct((Y * M_PER, N_PER), x.dtype),
        in_specs=[pl.BlockSpec(memory_space=pltpu.MemorySpace.HBM)],
        out_specs=pl.BlockSpec(memory_space=pltpu.MemorySpace.HBM),
        scratch_shapes=[
            pltpu.SemaphoreType.DMA,
            pltpu.SemaphoreType.DMA((3,)),
            pltpu.SemaphoreType.DMA((3,)),
            pltpu.SemaphoreType.DMA((3,)),
            pltpu.SemaphoreType.DMA((3,)),
            pltpu.SemaphoreType.DMA((9,)),
            pltpu.SemaphoreType.DMA((6,)),
        ],
        compiler_params=pltpu.CompilerParams(collective_id=0),
    )(x)


# device time: 561565 ns/iter; 1.3751x vs baseline; 1.1446x over previous
import functools

import jax
import jax.numpy as jnp
from jax import lax
from jax.experimental import pallas as pl
from jax.experimental.pallas import tpu as pltpu

Y = 4
QW = 256
M_PER = 4096
N_PER = 1024

S1_TARGETS = {0: [1], 1: [0, 2], 2: [1, 3], 3: [2]}
S2_TARGETS = {0: [], 1: [0], 2: [3], 3: []}


def kernel(x):
    def body(x_ref, out_ref, local_sem,
             ysend, yrecv, xsend, xrecv, zsend, zrecv):
        my_x = lax.axis_index("x")
        my_y = lax.axis_index("y")
        my_z = lax.axis_index("z")
        a = lax.rem(my_z + 2 * my_x, 4)

        def sig(sem, dev):
            pl.semaphore_signal(
                sem, inc=1, device_id=dev,
                device_id_type=pl.DeviceIdType.MESH,
            )

        def barrier_with_partners(sem):
            for d in range(1, Y):
                q = lax.rem(my_y + d, Y)
                sig(sem, (my_x, q, my_z))
            sig(sem, (1 - my_x, my_y, my_z))

            @pl.when(my_z > 0)
            def _():
                sig(sem, (my_x, my_y, my_z - 1))

            @pl.when(my_z < 3)
            def _():
                sig(sem, (my_x, my_y, my_z + 1))

            pl.semaphore_wait(sem, 5)

            @pl.when((my_z > 0) & (my_z < 3))
            def _():
                pl.semaphore_wait(sem, 1)

        barrier_with_partners(pltpu.get_barrier_semaphore())

        local = pltpu.make_async_copy(
            x_ref.at[:, pl.ds(my_y * N_PER, N_PER)],
            out_ref.at[pl.ds(my_y * M_PER, M_PER), :],
            local_sem,
        )
        local.start()

        yrd = []
        for d in range(1, Y):
            qdev = lax.rem(my_y + d, Y)
            r = pltpu.make_async_remote_copy(
                src_ref=x_ref.at[:, pl.ds(qdev * N_PER + a * QW, QW)],
                dst_ref=out_ref.at[pl.ds(my_y * M_PER, M_PER),
                                   pl.ds(a * QW, QW)],
                send_sem=ysend.at[d - 1],
                recv_sem=yrecv.at[d - 1],
                device_id=(my_x, qdev, my_z),
                device_id_type=pl.DeviceIdType.MESH,
            )
            r.start()
            yrd.append(r)

        def block_row(d):
            return lax.rem(my_y - d + Y, Y) * M_PER

        def start_z_sends(zz, which, d):
            p_row = block_row(d)
            if which == 1:
                quarter, targets, base = zz, S1_TARGETS[zz], 0
            else:
                quarter, targets, base = (zz + 2) % 4, S2_TARGETS[zz], 6
            for k, tz in enumerate(targets):
                slot = base + k * 3 + (d - 1)
                r = pltpu.make_async_remote_copy(
                    src_ref=out_ref.at[pl.ds(p_row, M_PER),
                                       pl.ds(quarter * QW, QW)],
                    dst_ref=out_ref.at[pl.ds(p_row, M_PER),
                                       pl.ds(quarter * QW, QW)],
                    send_sem=zsend.at[slot],
                    recv_sem=zrecv.at[(quarter // 2) * 3 + (d - 1)],
                    device_id=(my_x, my_y, tz),
                    device_id_type=pl.DeviceIdType.MESH,
                )
                r.start()

        for d in range(1, Y):
            yrd[d - 1].wait_recv()
            p_row = block_row(d)
            r = pltpu.make_async_remote_copy(
                src_ref=out_ref.at[pl.ds(p_row, M_PER),
                                   pl.ds(a * QW, QW)],
                dst_ref=out_ref.at[pl.ds(p_row, M_PER),
                                   pl.ds(a * QW, QW)],
                send_sem=xsend.at[d - 1],
                recv_sem=xrecv.at[d - 1],
                device_id=(1 - my_x, my_y, my_z),
                device_id_type=pl.DeviceIdType.MESH,
            )
            r.start()
            for zz in range(4):
                @pl.when((my_z == zz) & (my_x == 0))
                def _(zz=zz, d=d):
                    start_z_sends(zz, 1, d)

                @pl.when((my_z == zz) & (my_x == 1))
                def _(zz=zz, d=d):
                    start_z_sends(zz, 2, d)

        for d in range(1, Y):
            r = pltpu.make_async_remote_copy(
                src_ref=out_ref.at[pl.ds(0, M_PER), pl.ds(0, QW)],
                dst_ref=out_ref.at[pl.ds(0, M_PER), pl.ds(0, QW)],
                send_sem=xsend.at[d - 1],
                recv_sem=xrecv.at[d - 1],
                device_id=(1 - my_x, my_y, my_z),
                device_id_type=pl.DeviceIdType.MESH,
            )
            r.wait_recv()
            for zz in range(4):
                @pl.when((my_z == zz) & (my_x == 0))
                def _(zz=zz, d=d):
                    start_z_sends(zz, 2, d)

                @pl.when((my_z == zz) & (my_x == 1))
                def _(zz=zz, d=d):
                    start_z_sends(zz, 1, d)

        for qslot in range(2):
            for d in range(1, Y):
                r = pltpu.make_async_remote_copy(
                    src_ref=out_ref.at[pl.ds(0, M_PER), pl.ds(0, QW)],
                    dst_ref=out_ref.at[pl.ds(0, M_PER), pl.ds(0, QW)],
                    send_sem=zsend.at[0],
                    recv_sem=zrecv.at[qslot * 3 + (d - 1)],
                    device_id=(my_x, my_y, my_z),
                    device_id_type=pl.DeviceIdType.MESH,
                )
                r.wait_recv()

        for d in range(1, Y):
            yrd[d - 1].wait_send()
            r = pltpu.make_async_remote_copy(
                src_ref=out_ref.at[pl.ds(0, M_PER), pl.ds(0, QW)],
                dst_ref=out_ref.at[pl.ds(0, M_PER), pl.ds(0, QW)],
                send_sem=xsend.at[d - 1],
                recv_sem=xrecv.at[d - 1],
                device_id=(my_x, my_y, my_z),
                device_id_type=pl.DeviceIdType.MESH,
            )
            r.wait_send()

        for zz in range(4):
            n_slots = 3 * len(S1_TARGETS[zz]) + 3 * len(S2_TARGETS[zz])
            slots = (
                list(range(3 * len(S1_TARGETS[zz])))
                + list(range(6, 6 + 3 * len(S2_TARGETS[zz])))
            )
            assert len(slots) == n_slots

            @pl.when(my_z == zz)
            def _(slots=slots):
                for s in slots:
                    r = pltpu.make_async_remote_copy(
                        src_ref=out_ref.at[pl.ds(0, M_PER), pl.ds(0, QW)],
                        dst_ref=out_ref.at[pl.ds(0, M_PER), pl.ds(0, QW)],
                        send_sem=zsend.at[s],
                        recv_sem=zrecv.at[0],
                        device_id=(my_x, my_y, my_z),
                        device_id_type=pl.DeviceIdType.MESH,
                    )
                    r.wait_send()

        local.wait()

        @functools.partial(
            pl.run_scoped, exit_sem=pltpu.SemaphoreType.REGULAR
        )
        def _(exit_sem):
            barrier_with_partners(exit_sem)

    return pl.pallas_call(
        body,
        out_shape=jax.ShapeDtypeStruct((Y * M_PER, N_PER), x.dtype),
        in_specs=[pl.BlockSpec(memory_space=pltpu.MemorySpace.HBM)],
        out_specs=pl.BlockSpec(memory_space=pltpu.MemorySpace.HBM),
        scratch_shapes=[
            pltpu.SemaphoreType.DMA,
            pltpu.SemaphoreType.DMA((3,)),
            pltpu.SemaphoreType.DMA((3,)),
            pltpu.SemaphoreType.DMA((3,)),
            pltpu.SemaphoreType.DMA((3,)),
            pltpu.SemaphoreType.DMA((9,)),
            pltpu.SemaphoreType.DMA((6,)),
        ],
        compiler_params=pltpu.CompilerParams(collective_id=0),
    )(x)


# device time: 561480 ns/iter; 1.3753x vs baseline; 1.0002x over previous
import functools

import jax
import jax.numpy as jnp
from jax import lax
from jax.experimental import pallas as pl
from jax.experimental.pallas import tpu as pltpu

Y = 4
QW = 256
M_PER = 4096
N_PER = 1024


def kernel(x):
    def body(x_ref, out_ref, local_sem,
             ysend, yrecv, xsend, xrecv, zsend, zrecv):
        my_x = lax.axis_index("x")
        my_y = lax.axis_index("y")
        my_z = lax.axis_index("z")
        a = lax.rem(my_z + 2 * my_x, 4)
        b = lax.rem(a + 2, 4)
        zp = my_z ^ 1

        def sig(sem, dev):
            pl.semaphore_signal(
                sem, inc=1, device_id=dev,
                device_id_type=pl.DeviceIdType.MESH,
            )

        def barrier_with_partners(sem):
            for d in range(1, Y):
                q = lax.rem(my_y + d, Y)
                sig(sem, (my_x, q, my_z))
            sig(sem, (1 - my_x, my_y, my_z))
            sig(sem, (my_x, my_y, zp))
            pl.semaphore_wait(sem, 5)

        barrier_with_partners(pltpu.get_barrier_semaphore())

        local = pltpu.make_async_copy(
            x_ref.at[:, pl.ds(my_y * N_PER, N_PER)],
            out_ref.at[pl.ds(my_y * M_PER, M_PER), :],
            local_sem,
        )
        local.start()

        yrd = []
        for d in range(1, Y):
            qdev = lax.rem(my_y + d, Y)
            r = pltpu.make_async_remote_copy(
                src_ref=x_ref.at[:, pl.ds(qdev * N_PER + a * QW, QW)],
                dst_ref=out_ref.at[pl.ds(my_y * M_PER, M_PER),
                                   pl.ds(a * QW, QW)],
                send_sem=ysend.at[d - 1],
                recv_sem=yrecv.at[d - 1],
                device_id=(my_x, qdev, my_z),
                device_id_type=pl.DeviceIdType.MESH,
            )
            r.start()
            yrd.append(r)

        def block_row(d):
            return lax.rem(my_y - d + Y, Y) * M_PER

        def quarter_copy(d, quarter, send_ref, recv_ref, dev):
            p_row = block_row(d)
            return pltpu.make_async_remote_copy(
                src_ref=out_ref.at[pl.ds(p_row, M_PER),
                                   pl.ds(quarter * QW, QW)],
                dst_ref=out_ref.at[pl.ds(p_row, M_PER),
                                   pl.ds(quarter * QW, QW)],
                send_sem=send_ref,
                recv_sem=recv_ref,
                device_id=dev,
                device_id_type=pl.DeviceIdType.MESH,
            )

        xrd = []
        zrd = []
        for d in range(1, Y):
            yrd[d - 1].wait_recv()
            r = quarter_copy(d, a, xsend.at[d - 1], xrecv.at[d - 1],
                             (1 - my_x, my_y, my_z))
            r.start()
            xrd.append(r)
            r = quarter_copy(d, a, zsend.at[d - 1], zrecv.at[d - 1],
                             (my_x, my_y, zp))
            r.start()
            zrd.append(r)

        for d in range(1, Y):
            xrd[d - 1].wait_recv()
            r = quarter_copy(d, b, zsend.at[3 + d - 1],
                             zrecv.at[3 + d - 1], (my_x, my_y, zp))
            r.start()
            zrd.append(r)

        for s in range(6):
            r = pltpu.make_async_remote_copy(
                src_ref=out_ref.at[pl.ds(0, M_PER), pl.ds(0, QW)],
                dst_ref=out_ref.at[pl.ds(0, M_PER), pl.ds(0, QW)],
                send_sem=zsend.at[0],
                recv_sem=zrecv.at[s],
                device_id=(my_x, my_y, my_z),
                device_id_type=pl.DeviceIdType.MESH,
            )
            r.wait_recv()

        for r in yrd:
            r.wait_send()
        for r in xrd:
            r.wait_send()
        for r in zrd:
            r.wait_send()

        local.wait()

        @functools.partial(
            pl.run_scoped, exit_sem=pltpu.SemaphoreType.REGULAR
        )
        def _(exit_sem):
            barrier_with_partners(exit_sem)

    return pl.pallas_call(
        body,
        out_shape=jax.ShapeDtypeStruct((Y * M_PER, N_PER), x.dtype),
        in_specs=[pl.BlockSpec(memory_space=pltpu.MemorySpace.HBM)],
        out_specs=pl.BlockSpec(memory_space=pltpu.MemorySpace.HBM),
        scratch_shapes=[
            pltpu.SemaphoreType.DMA,
            pltpu.SemaphoreType.DMA((3,)),
            pltpu.SemaphoreType.DMA((3,)),
            pltpu.SemaphoreType.DMA((3,)),
            pltpu.SemaphoreType.DMA((3,)),
            pltpu.SemaphoreType.DMA((6,)),
            pltpu.SemaphoreType.DMA((6,)),
        ],
        compiler_params=pltpu.CompilerParams(collective_id=0),
    )(x)
